# baseline (speedup 1.0000x reference)
import jax
import jax.numpy as jnp
from jax import lax
from jax.experimental import pallas as pl

EMB_D = 64
USR_D = 16
NEXP = 64
BLK = 4096


def _gate_body(h_ref, u_ref, wg_ref, bg_ref, wb_ref, bb_ref, wl_ref,
               bl_ref, out_ref):
    u = u_ref[...]
    h = h_ref[...]
    dn = (((1,), (1,)), ((), ()))
    gamma = lax.dot_general(u, wg_ref[...], dn,
                            preferred_element_type=jnp.float32)
    gamma = gamma + bg_ref[...]
    beta = lax.dot_general(u, wb_ref[...], dn,
                           preferred_element_type=jnp.float32)
    beta = beta + bb_ref[...]
    h_t = h * (1.0 + gamma) + beta
    logits = lax.dot_general(h_t, wl_ref[...], dn,
                             preferred_element_type=jnp.float32)
    logits = logits + bl_ref[...]

    row = lax.broadcasted_iota(jnp.int32, (NEXP, NEXP), 0)
    col = lax.broadcasted_iota(jnp.int32, (NEXP, NEXP), 1)
    lt = (row <= col).astype(jnp.float32)

    m1 = jnp.max(logits, axis=1, keepdims=True)
    eq1 = logits == m1
    cs1 = jnp.dot(eq1.astype(jnp.float32), lt,
                  preferred_element_type=jnp.float32)
    mask1 = eq1 & (cs1 == 1.0)
    l2 = jnp.where(mask1, -jnp.inf, logits)
    m2 = jnp.max(l2, axis=1, keepdims=True)
    eq2 = l2 == m2
    cs2 = jnp.dot(eq2.astype(jnp.float32), lt,
                  preferred_element_type=jnp.float32)
    mask2 = eq2 & (cs2 == 1.0)

    e = jnp.exp(m2 - m1)
    denom = 1.0 + e
    p1 = 1.0 / denom
    p2 = e / denom
    out_ref[...] = jnp.where(mask1, p1, jnp.where(mask2, p2, 0.0))


def kernel(h, u, Wg, bg, Wb, bb, Wl, bl):
    n = h.shape[0]
    grid = (n // BLK,)
    bg2 = bg[None, :]
    bb2 = bb[None, :]
    bl2 = bl[None, :]
    return pl.pallas_call(
        _gate_body,
        grid=grid,
        in_specs=[
            pl.BlockSpec((BLK, EMB_D), lambda i: (i, 0)),
            pl.BlockSpec((BLK, USR_D), lambda i: (i, 0)),
            pl.BlockSpec((EMB_D, USR_D), lambda i: (0, 0)),
            pl.BlockSpec((1, EMB_D), lambda i: (0, 0)),
            pl.BlockSpec((EMB_D, USR_D), lambda i: (0, 0)),
            pl.BlockSpec((1, EMB_D), lambda i: (0, 0)),
            pl.BlockSpec((NEXP, EMB_D), lambda i: (0, 0)),
            pl.BlockSpec((1, NEXP), lambda i: (0, 0)),
        ],
        out_specs=pl.BlockSpec((BLK, NEXP), lambda i: (i, 0)),
        out_shape=jax.ShapeDtypeStruct((n, NEXP), jnp.float32),
    )(h, u, Wg, bg2, Wb, bb2, Wl, bl2)


# speedup vs baseline: 1.0212x; 1.0212x over previous
import functools

import jax
import jax.numpy as jnp
from jax import lax
from jax.experimental import pallas as pl
from jax.experimental.pallas import tpu as pltpu
from jax.experimental.pallas import tpu_sc as plsc

EMB_D = 64
USR_D = 16
NEXP = 64
BLK = 4096


def _gate_body(h_ref, u_ref, wg_ref, bg_ref, wb_ref, bb_ref, wl_ref,
               bl_ref, out_ref):
    u = u_ref[...]
    h = h_ref[...]
    gamma = jnp.dot(u, wg_ref[...], preferred_element_type=jnp.float32)
    gamma = gamma + bg_ref[...]
    beta = jnp.dot(u, wb_ref[...], preferred_element_type=jnp.float32)
    beta = beta + bb_ref[...]
    h_t = h * (1.0 + gamma) + beta
    logits = jnp.dot(h_t, wl_ref[...], preferred_element_type=jnp.float32)
    logits = logits + bl_ref[...]

    row = lax.broadcasted_iota(jnp.int32, (NEXP, NEXP), 0)
    col = lax.broadcasted_iota(jnp.int32, (NEXP, NEXP), 1)
    lt = (row <= col).astype(jnp.float32)

    m1 = jnp.max(logits, axis=1, keepdims=True)
    eq1 = logits == m1
    cs1 = jnp.dot(eq1.astype(jnp.float32), lt,
                  preferred_element_type=jnp.float32)
    mask1 = eq1 & (cs1 == 1.0)
    l2 = jnp.where(mask1, -jnp.inf, logits)
    m2 = jnp.max(l2, axis=1, keepdims=True)
    eq2 = l2 == m2
    cs2 = jnp.dot(eq2.astype(jnp.float32), lt,
                  preferred_element_type=jnp.float32)
    mask2 = eq2 & (cs2 == 1.0)

    e = jnp.exp(m2 - m1)
    denom = 1.0 + e
    p1 = 1.0 / denom
    p2 = e / denom
    out_ref[...] = jnp.where(mask1, p1, jnp.where(mask2, p2, 0.0))


_N_TOK = 32768
_SC_ROWS = _N_TOK // 32


@functools.partial(
    pl.kernel,
    out_type=jax.ShapeDtypeStruct((_N_TOK, EMB_D), jnp.float32),
    mesh=plsc.VectorSubcoreMesh(core_axis_name="c", subcore_axis_name="s"),
    scratch_types=[pltpu.VMEM((_SC_ROWS, EMB_D), jnp.float32)],
)
def _sc_copy(h_hbm, out_hbm, buf):
    wid = lax.axis_index("s") * 2 + lax.axis_index("c")
    base = wid * _SC_ROWS
    pltpu.sync_copy(h_hbm.at[pl.ds(base, _SC_ROWS)], buf)
    pltpu.sync_copy(buf, out_hbm.at[pl.ds(base, _SC_ROWS)])


def kernel(h, u, Wg, bg, Wb, bb, Wl, bl):
    n = h.shape[0]
    grid = (n // BLK,)
    bg2 = bg[None, :]
    bb2 = bb[None, :]
    bl2 = bl[None, :]
    w = pl.pallas_call(
        _gate_body,
        grid=grid,
        in_specs=[
            pl.BlockSpec((BLK, EMB_D), lambda i: (i, 0)),
            pl.BlockSpec((BLK, USR_D), lambda i: (i, 0)),
            pl.BlockSpec((USR_D, EMB_D), lambda i: (0, 0)),
            pl.BlockSpec((1, EMB_D), lambda i: (0, 0)),
            pl.BlockSpec((USR_D, EMB_D), lambda i: (0, 0)),
            pl.BlockSpec((1, EMB_D), lambda i: (0, 0)),
            pl.BlockSpec((EMB_D, NEXP), lambda i: (0, 0)),
            pl.BlockSpec((1, NEXP), lambda i: (0, 0)),
        ],
        out_specs=pl.BlockSpec((BLK, NEXP), lambda i: (i, 0)),
        out_shape=jax.ShapeDtypeStruct((n, NEXP), jnp.float32),
    )(h, u, Wg.T, bg2, Wb.T, bb2, Wl.T, bl2)
    probe = _sc_copy(h)
    w, _ = lax.optimization_barrier((w, probe))
    return w
